# Initial kernel scaffold; baseline (speedup 1.0000x reference)
import functools

import jax
import jax.numpy as jnp
from jax import lax
from jax.experimental import pallas as pl
from jax.experimental.pallas import tpu as pltpu
from jax.experimental.pallas import tpu_sc as plsc

N = 10000
E = 320000
F = 128
H = 16
C = 4

NC = 2
NS = 16
NW = NC * NS
EPW = E // NW
CHUNK = 80
NCH = EPW // CHUNK
RPS = N // NS

_mesh = plsc.VectorSubcoreMesh(
    core_axis_name="c", subcore_axis_name="s", num_cores=NC, num_subcores=NS
)

_ZERO_ROW = jnp.zeros((H,), jnp.float32)


def _stripe_out(zb_v, acc_sh, out_hbm, cid, sid):
    pltpu.sync_copy(acc_sh.at[pl.ds(sid * RPS, RPS)], zb_v)
    pltpu.sync_copy(zb_v, out_hbm.at[cid, pl.ds(sid * RPS, RPS)])


def _zero_acc(zb_v, acc_sh, sid):
    def zfill(i, c):
        zb_v[i, :] = _ZERO_ROW
        return c
    lax.fori_loop(0, RPS, zfill, 0)
    pltpu.sync_copy(zb_v, acc_sh.at[pl.ds(sid * RPS, RPS)])


@functools.partial(
    pl.kernel,
    out_type=jax.ShapeDtypeStruct((NC, N, H), jnp.float32),
    mesh=_mesh,
    scratch_types=[
        pltpu.VMEM((NCH, CHUNK), jnp.int32),
        pltpu.VMEM((CHUNK, H), jnp.float32),
        pltpu.VMEM((RPS, H), jnp.float32),
        pltpu.VMEM_SHARED((N, H), jnp.float32),
    ],
)
def _deg_kernel(col_hbm, out_hbm, coli_v, ones_v, zb_v, acc_sh):
    cid = lax.axis_index("c")
    sid = lax.axis_index("s")
    base = (cid * NS + sid) * EPW

    one_row = jnp.full((H,), 1.0, jnp.float32)

    def fill(i, c):
        ones_v[i, :] = one_row
        return c
    lax.fori_loop(0, CHUNK, fill, 0)

    _zero_acc(zb_v, acc_sh, sid)

    def ld(j, c):
        pltpu.sync_copy(col_hbm.at[pl.ds(base + j * CHUNK, CHUNK)], coli_v.at[j])
        return c
    lax.fori_loop(0, NCH, ld, 0)

    plsc.subcore_barrier()

    def sc(j, c):
        pltpu.sync_copy(ones_v, acc_sh.at[coli_v.at[j]], add=True)
        return c
    lax.fori_loop(0, NCH, sc, 0)

    plsc.subcore_barrier()
    _stripe_out(zb_v, acc_sh, out_hbm, cid, sid)


@functools.partial(
    pl.kernel,
    out_type=jax.ShapeDtypeStruct((NC, N, H), jnp.float32),
    mesh=_mesh,
    scratch_types=[
        pltpu.VMEM((NCH, CHUNK), jnp.int32),
        pltpu.VMEM((NCH, CHUNK), jnp.int32),
        pltpu.VMEM((CHUNK, H), jnp.float32),
        pltpu.VMEM((RPS, H), jnp.float32),
        pltpu.VMEM_SHARED((N, H), jnp.float32),
        pltpu.SemaphoreType.DMA,
    ],
)
def _prop_kernel(g_hbm, row_hbm, col_hbm, out_hbm,
                 rowi_v, coli_v, rows_v, zb_v, acc_sh, sem):
    cid = lax.axis_index("c")
    sid = lax.axis_index("s")
    base = (cid * NS + sid) * EPW

    _zero_acc(zb_v, acc_sh, sid)

    def ld(j, c):
        pltpu.sync_copy(row_hbm.at[pl.ds(base + j * CHUNK, CHUNK)], rowi_v.at[j])
        pltpu.sync_copy(col_hbm.at[pl.ds(base + j * CHUNK, CHUNK)], coli_v.at[j])
        return c
    lax.fori_loop(0, NCH, ld, 0)

    plsc.subcore_barrier()

    def step(j, c):
        pltpu.async_copy(g_hbm.at[rowi_v.at[j]], rows_v, sem).wait()
        pltpu.sync_copy(rows_v, acc_sh.at[coli_v.at[j]], add=True)
        return c
    lax.fori_loop(0, NCH, step, 0)

    plsc.subcore_barrier()
    _stripe_out(zb_v, acc_sh, out_hbm, cid, sid)


BN = 1000


def _mm(a, b):
    return lax.dot_general(a, b, (((1,), (0,)), ((), ())),
                           preferred_element_type=jnp.float32)


def _dinv(dacc_ref):
    return lax.rsqrt(dacc_ref[0] + dacc_ref[1] + 1.0)


def _tc1_body(x_ref, w1_ref, dacc_ref, g_ref):
    g_ref[...] = _mm(x_ref[...], w1_ref[...]) * _dinv(dacc_ref)


_tc1 = pl.pallas_call(
    _tc1_body,
    grid=(N // BN,),
    in_specs=[
        pl.BlockSpec((BN, F), lambda i: (i, 0)),
        pl.BlockSpec((F, H), lambda i: (0, 0)),
        pl.BlockSpec((NC, BN, H), lambda i: (0, i, 0)),
    ],
    out_specs=pl.BlockSpec((BN, H), lambda i: (i, 0)),
    out_shape=jax.ShapeDtypeStruct((N, H), jnp.float32),
)


def _tc2_body(acc_ref, g_ref, dacc_ref, w2_ref, b1_ref, out_ref):
    dinv = _dinv(dacc_ref)
    s1 = jnp.maximum(dinv * (acc_ref[0] + acc_ref[1] + g_ref[...]) + b1_ref[...],
                     0.0)
    out_ref[...] = _mm(s1, w2_ref[...]) * dinv


_tc2 = pl.pallas_call(
    _tc2_body,
    grid=(N // BN,),
    in_specs=[
        pl.BlockSpec((NC, BN, H), lambda i: (0, i, 0)),
        pl.BlockSpec((BN, H), lambda i: (i, 0)),
        pl.BlockSpec((NC, BN, H), lambda i: (0, i, 0)),
        pl.BlockSpec((H, H), lambda i: (0, 0)),
        pl.BlockSpec((1, H), lambda i: (0, 0)),
    ],
    out_specs=pl.BlockSpec((BN, H), lambda i: (i, 0)),
    out_shape=jax.ShapeDtypeStruct((N, H), jnp.float32),
)


def _tc3_body(acc_ref, g_ref, dacc_ref, wfc_ref, b2_ref, bfc_ref, out_ref):
    dinv = _dinv(dacc_ref)
    s2 = jnp.maximum(dinv * (acc_ref[0] + acc_ref[1] + g_ref[...]) + b2_ref[...],
                     0.0)
    out_ref[...] = _mm(s2, wfc_ref[...]) + bfc_ref[...]


_tc3 = pl.pallas_call(
    _tc3_body,
    grid=(N // BN,),
    in_specs=[
        pl.BlockSpec((NC, BN, H), lambda i: (0, i, 0)),
        pl.BlockSpec((BN, H), lambda i: (i, 0)),
        pl.BlockSpec((NC, BN, H), lambda i: (0, i, 0)),
        pl.BlockSpec((H, C), lambda i: (0, 0)),
        pl.BlockSpec((1, H), lambda i: (0, 0)),
        pl.BlockSpec((1, C), lambda i: (0, 0)),
    ],
    out_specs=pl.BlockSpec((BN, C), lambda i: (i, 0)),
    out_shape=jax.ShapeDtypeStruct((N, C), jnp.float32),
)


@jax.jit
def kernel(x, edge_index, W1, b1, W2, b2, Wfc, bfc):
    row = edge_index[0].astype(jnp.int32)
    col = edge_index[1].astype(jnp.int32)
    dacc = _deg_kernel(col)
    g1 = _tc1(x, W1, dacc)
    acc1 = _prop_kernel(g1, row, col)
    g2 = _tc2(acc1, g1, dacc, W2, b1.reshape(1, H))
    acc2 = _prop_kernel(g2, row, col)
    return _tc3(acc2, g2, dacc, Wfc, b2.reshape(1, H), bfc.reshape(1, C))


# speedup vs baseline: 16.4910x; 16.4910x over previous
import functools

import jax
import jax.numpy as jnp
from jax import lax
from jax.experimental import pallas as pl
from jax.experimental.pallas import tpu as pltpu
from jax.experimental.pallas import tpu_sc as plsc

N = 10000
E = 320000
F = 128
H = 16
C = 4

NC = 2
NS = 16
NW = NC * NS
EPW = E // NW
CHUNK = 80
NCH = EPW // CHUNK
RPS = N // NS

_mesh = plsc.VectorSubcoreMesh(
    core_axis_name="c", subcore_axis_name="s", num_cores=NC, num_subcores=NS
)

_sc_params = pltpu.CompilerParams(use_tc_tiling_on_sc=False)

def _stripe_out(zb_v, acc_sh, out_hbm, cid, sid):
    pltpu.sync_copy(acc_sh.at[pl.ds(sid * RPS, RPS)], zb_v)
    pltpu.sync_copy(zb_v, out_hbm.at[cid, sid])


def _zero_acc(zb_v, acc_sh, sid):
    zero_row = jnp.zeros((H,), jnp.float32)

    def zfill(i, c):
        zb_v[i, :] = zero_row
        return c
    lax.fori_loop(0, RPS, zfill, 0)
    pltpu.sync_copy(zb_v, acc_sh.at[pl.ds(sid * RPS, RPS)])


@functools.partial(
    pl.kernel,
    out_type=jax.ShapeDtypeStruct((NC, NS, RPS, H), jnp.float32),
    mesh=_mesh,
    scratch_types=[
        pltpu.VMEM((NCH, CHUNK), jnp.int32),
        pltpu.VMEM((CHUNK, H), jnp.float32),
        pltpu.VMEM((RPS, H), jnp.float32),
        pltpu.VMEM_SHARED((N, H), jnp.float32),
    ],
    compiler_params=_sc_params,
)
def _deg_kernel(col_hbm, out_hbm, coli_v, ones_v, zb_v, acc_sh):
    cid = lax.axis_index("c")
    sid = lax.axis_index("s")
    base = (cid * NS + sid) * EPW

    one_row = jnp.full((H,), 1.0, jnp.float32)

    def fill(i, c):
        ones_v[i, :] = one_row
        return c
    lax.fori_loop(0, CHUNK, fill, 0)

    _zero_acc(zb_v, acc_sh, sid)

    def ld(j, c):
        pltpu.sync_copy(col_hbm.at[pl.ds(base + j * CHUNK, CHUNK)], coli_v.at[j])
        return c
    lax.fori_loop(0, NCH, ld, 0)

    plsc.subcore_barrier()

    def sc(j, c):
        pltpu.sync_copy(ones_v, acc_sh.at[coli_v.at[j]], add=True)
        return c
    lax.fori_loop(0, NCH, sc, 0)

    plsc.subcore_barrier()
    _stripe_out(zb_v, acc_sh, out_hbm, cid, sid)


@functools.partial(
    pl.kernel,
    out_type=jax.ShapeDtypeStruct((NC, NS, RPS, H), jnp.float32),
    mesh=_mesh,
    scratch_types=[
        pltpu.VMEM((NCH, CHUNK), jnp.int32),
        pltpu.VMEM((NCH, CHUNK), jnp.int32),
        pltpu.VMEM((CHUNK, H), jnp.float32),
        pltpu.VMEM((RPS, H), jnp.float32),
        pltpu.VMEM_SHARED((N, H), jnp.float32),
        pltpu.SemaphoreType.DMA,
    ],
    compiler_params=_sc_params,
)
def _prop_kernel(g_hbm, row_hbm, col_hbm, out_hbm,
                 rowi_v, coli_v, rows_v, zb_v, acc_sh, sem):
    cid = lax.axis_index("c")
    sid = lax.axis_index("s")
    base = (cid * NS + sid) * EPW

    _zero_acc(zb_v, acc_sh, sid)

    def ld(j, c):
        pltpu.sync_copy(row_hbm.at[pl.ds(base + j * CHUNK, CHUNK)], rowi_v.at[j])
        pltpu.sync_copy(col_hbm.at[pl.ds(base + j * CHUNK, CHUNK)], coli_v.at[j])
        return c
    lax.fori_loop(0, NCH, ld, 0)

    plsc.subcore_barrier()

    def step(j, c):
        pltpu.async_copy(g_hbm.at[rowi_v.at[j]], rows_v, sem).wait()
        pltpu.sync_copy(rows_v, acc_sh.at[coli_v.at[j]], add=True)
        return c
    lax.fori_loop(0, NCH, step, 0)

    plsc.subcore_barrier()
    _stripe_out(zb_v, acc_sh, out_hbm, cid, sid)


BN = 1000


def _mm(a, b):
    return lax.dot_general(a, b, (((1,), (0,)), ((), ())),
                           preferred_element_type=jnp.float32)


def _dinv(dacc_ref):
    return lax.rsqrt(dacc_ref[0] + dacc_ref[1] + 1.0)


def _tc1_body(x_ref, w1_ref, dacc_ref, g_ref):
    g_ref[...] = _mm(x_ref[...], w1_ref[...]) * _dinv(dacc_ref)


_tc1 = pl.pallas_call(
    _tc1_body,
    grid=(N // BN,),
    in_specs=[
        pl.BlockSpec((BN, F), lambda i: (i, 0)),
        pl.BlockSpec((F, H), lambda i: (0, 0)),
        pl.BlockSpec((NC, BN, H), lambda i: (0, i, 0)),
    ],
    out_specs=pl.BlockSpec((BN, H), lambda i: (i, 0)),
    out_shape=jax.ShapeDtypeStruct((N, H), jnp.float32),
)


def _tc2_body(acc_ref, g_ref, dacc_ref, w2_ref, b1_ref, out_ref):
    dinv = _dinv(dacc_ref)
    s1 = jnp.maximum(dinv * (acc_ref[0] + acc_ref[1] + g_ref[...]) + b1_ref[...],
                     0.0)
    out_ref[...] = _mm(s1, w2_ref[...]) * dinv


_tc2 = pl.pallas_call(
    _tc2_body,
    grid=(N // BN,),
    in_specs=[
        pl.BlockSpec((NC, BN, H), lambda i: (0, i, 0)),
        pl.BlockSpec((BN, H), lambda i: (i, 0)),
        pl.BlockSpec((NC, BN, H), lambda i: (0, i, 0)),
        pl.BlockSpec((H, H), lambda i: (0, 0)),
        pl.BlockSpec((1, H), lambda i: (0, 0)),
    ],
    out_specs=pl.BlockSpec((BN, H), lambda i: (i, 0)),
    out_shape=jax.ShapeDtypeStruct((N, H), jnp.float32),
)


def _tc3_body(acc_ref, g_ref, dacc_ref, wfc_ref, b2_ref, bfc_ref, out_ref):
    dinv = _dinv(dacc_ref)
    s2 = jnp.maximum(dinv * (acc_ref[0] + acc_ref[1] + g_ref[...]) + b2_ref[...],
                     0.0)
    out_ref[...] = _mm(s2, wfc_ref[...]) + bfc_ref[...]


_tc3 = pl.pallas_call(
    _tc3_body,
    grid=(N // BN,),
    in_specs=[
        pl.BlockSpec((NC, BN, H), lambda i: (0, i, 0)),
        pl.BlockSpec((BN, H), lambda i: (i, 0)),
        pl.BlockSpec((NC, BN, H), lambda i: (0, i, 0)),
        pl.BlockSpec((H, C), lambda i: (0, 0)),
        pl.BlockSpec((1, H), lambda i: (0, 0)),
        pl.BlockSpec((1, C), lambda i: (0, 0)),
    ],
    out_specs=pl.BlockSpec((BN, C), lambda i: (i, 0)),
    out_shape=jax.ShapeDtypeStruct((N, C), jnp.float32),
)


@jax.jit
def kernel(x, edge_index, W1, b1, W2, b2, Wfc, bfc):
    row = edge_index[0].astype(jnp.int32)
    col = edge_index[1].astype(jnp.int32)
    dacc = _deg_kernel(col).reshape(NC, N, H)
    g1 = _tc1(x, W1, dacc)
    acc1 = _prop_kernel(g1, row, col).reshape(NC, N, H)
    g2 = _tc2(acc1, g1, dacc, W2, b1.reshape(1, H))
    acc2 = _prop_kernel(g2, row, col).reshape(NC, N, H)
    return _tc3(acc2, g2, dacc, Wfc, b2.reshape(1, H), bfc.reshape(1, C))


# speedup vs baseline: 49.2260x; 2.9850x over previous
import functools

import jax
import jax.numpy as jnp
from jax import lax
from jax.experimental import pallas as pl
from jax.experimental.pallas import tpu as pltpu
from jax.experimental.pallas import tpu_sc as plsc

N = 10000
E = 320000
F = 128
H = 16
C = 4

NC = 2
NS = 16
NW = NC * NS

CHUNK = 80
NCH = 125
RPS = N // NS

NBUF = 5
AHEAD = 4

_mesh = plsc.VectorSubcoreMesh(
    core_axis_name="c", subcore_axis_name="s", num_cores=NC, num_subcores=NS
)

_sc_params = pltpu.CompilerParams(use_tc_tiling_on_sc=False)


def _stripe_out(zb_v, acc_sh, out_hbm, cid, sid):
    pltpu.sync_copy(acc_sh.at[pl.ds(sid * RPS, RPS)], zb_v)
    pltpu.sync_copy(zb_v, out_hbm.at[cid, sid])


def _zero_acc(zb_v, acc_sh, sid):
    zero_row = jnp.zeros((H,), jnp.float32)

    def zfill(i, c):
        zb_v[i, :] = zero_row
        return c
    lax.fori_loop(0, RPS, zfill, 0)
    pltpu.sync_copy(zb_v, acc_sh.at[pl.ds(sid * RPS, RPS)])


@functools.partial(
    pl.kernel,
    out_type=jax.ShapeDtypeStruct((NC, NS, RPS, H), jnp.float32),
    mesh=_mesh,
    scratch_types=[
        pltpu.VMEM((NCH, CHUNK), jnp.int32),
        pltpu.VMEM((CHUNK, H), jnp.float32),
        pltpu.VMEM((RPS, H), jnp.float32),
        pltpu.VMEM_SHARED((N, H), jnp.float32),
        pltpu.SemaphoreType.DMA,
    ],
    compiler_params=_sc_params,
)
def _deg_kernel(col_hbm, out_hbm, coli_v, ones_v, zb_v, acc_sh, sem_s):
    cid = lax.axis_index("c")
    sid = lax.axis_index("s")
    wid = cid * NS + sid

    one_row = jnp.full((H,), 1.0, jnp.float32)

    def fill(i, c):
        ones_v[i, :] = one_row
        return c
    lax.fori_loop(0, CHUNK, fill, 0)

    _zero_acc(zb_v, acc_sh, sid)
    pltpu.sync_copy(col_hbm.at[wid], coli_v)
    plsc.subcore_barrier()

    def scat(j):
        return pltpu.make_async_copy(ones_v, acc_sh.at[coli_v.at[j]], sem_s)

    LAG = 8

    def sc(j, c):
        scat(j).start(add=True)

        @pl.when(j >= LAG)
        def _():
            scat(j - LAG).wait()
        return c
    lax.fori_loop(0, NCH, sc, 0)

    def drain(j, c):
        scat(NCH - LAG + j).wait()
        return c
    lax.fori_loop(0, LAG, drain, 0)

    plsc.subcore_barrier()
    _stripe_out(zb_v, acc_sh, out_hbm, cid, sid)


@functools.partial(
    pl.kernel,
    out_type=jax.ShapeDtypeStruct((NC, NS, RPS, H), jnp.float32),
    mesh=_mesh,
    scratch_types=[
        pltpu.VMEM((NCH, CHUNK), jnp.int32),
        pltpu.VMEM((NCH, CHUNK), jnp.int32),
        pltpu.VMEM((CHUNK, H), jnp.float32),
        pltpu.VMEM((CHUNK, H), jnp.float32),
        pltpu.VMEM((CHUNK, H), jnp.float32),
        pltpu.VMEM((CHUNK, H), jnp.float32),
        pltpu.VMEM((CHUNK, H), jnp.float32),
        pltpu.VMEM((RPS, H), jnp.float32),
        pltpu.VMEM_SHARED((N, H), jnp.float32),
        pltpu.SemaphoreType.DMA,
        pltpu.SemaphoreType.DMA,
    ],
    compiler_params=_sc_params,
)
def _prop_kernel(g_hbm, row_hbm, col_hbm, out_hbm,
                 rowi_v, coli_v, r0, r1, r2, r3, r4, zb_v, acc_sh,
                 sem_g, sem_s):
    cid = lax.axis_index("c")
    sid = lax.axis_index("s")
    wid = cid * NS + sid
    bufs = (r0, r1, r2, r3, r4)

    _zero_acc(zb_v, acc_sh, sid)
    pltpu.sync_copy(row_hbm.at[wid], rowi_v)
    pltpu.sync_copy(col_hbm.at[wid], coli_v)
    plsc.subcore_barrier()

    def gat(j, b):
        return pltpu.make_async_copy(g_hbm.at[rowi_v.at[j]], bufs[b], sem_g)

    def scat(j, b):
        return pltpu.make_async_copy(bufs[b], acc_sh.at[coli_v.at[j]], sem_s)

    for j in range(AHEAD):
        gat(j, j).start()

    def step(g, c):
        for b in range(NBUF):
            j = g * NBUF + b
            bp = (b + AHEAD) % NBUF
            if b > 0:
                scat(j - 1, bp).wait()
            else:
                @pl.when(j >= 1)
                def _():
                    scat(j - 1, bp).wait()

            @pl.when(j + AHEAD < NCH)
            def _():
                gat(j + AHEAD, bp).start()

            gat(j, b).wait()
            scat(j, b).start(add=True)
        return c
    lax.fori_loop(0, NCH // NBUF, step, 0)

    scat(NCH - 1, (NCH - 1) % NBUF).wait()

    plsc.subcore_barrier()
    _stripe_out(zb_v, acc_sh, out_hbm, cid, sid)


BN = 1000


def _mm(a, b):
    return lax.dot_general(a, b, (((1,), (0,)), ((), ())),
                           preferred_element_type=jnp.float32)


def _dinv(dacc_ref):
    return lax.rsqrt(dacc_ref[0] + dacc_ref[1] + 1.0)


def _tc1_body(x_ref, w1_ref, dacc_ref, g_ref):
    g_ref[...] = _mm(x_ref[...], w1_ref[...]) * _dinv(dacc_ref)


_tc1 = pl.pallas_call(
    _tc1_body,
    grid=(N // BN,),
    in_specs=[
        pl.BlockSpec((BN, F), lambda i: (i, 0)),
        pl.BlockSpec((F, H), lambda i: (0, 0)),
        pl.BlockSpec((NC, BN, H), lambda i: (0, i, 0)),
    ],
    out_specs=pl.BlockSpec((BN, H), lambda i: (i, 0)),
    out_shape=jax.ShapeDtypeStruct((N, H), jnp.float32),
)


def _tc2_body(acc_ref, g_ref, dacc_ref, w2_ref, b1_ref, out_ref):
    dinv = _dinv(dacc_ref)
    s1 = jnp.maximum(dinv * (acc_ref[0] + acc_ref[1] + g_ref[...]) + b1_ref[...],
                     0.0)
    out_ref[...] = _mm(s1, w2_ref[...]) * dinv


_tc2 = pl.pallas_call(
    _tc2_body,
    grid=(N // BN,),
    in_specs=[
        pl.BlockSpec((NC, BN, H), lambda i: (0, i, 0)),
        pl.BlockSpec((BN, H), lambda i: (i, 0)),
        pl.BlockSpec((NC, BN, H), lambda i: (0, i, 0)),
        pl.BlockSpec((H, H), lambda i: (0, 0)),
        pl.BlockSpec((1, H), lambda i: (0, 0)),
    ],
    out_specs=pl.BlockSpec((BN, H), lambda i: (i, 0)),
    out_shape=jax.ShapeDtypeStruct((N, H), jnp.float32),
)


def _tc3_body(acc_ref, g_ref, dacc_ref, wfc_ref, b2_ref, bfc_ref, out_ref):
    dinv = _dinv(dacc_ref)
    s2 = jnp.maximum(dinv * (acc_ref[0] + acc_ref[1] + g_ref[...]) + b2_ref[...],
                     0.0)
    out_ref[...] = _mm(s2, wfc_ref[...]) + bfc_ref[...]


_tc3 = pl.pallas_call(
    _tc3_body,
    grid=(N // BN,),
    in_specs=[
        pl.BlockSpec((NC, BN, H), lambda i: (0, i, 0)),
        pl.BlockSpec((BN, H), lambda i: (i, 0)),
        pl.BlockSpec((NC, BN, H), lambda i: (0, i, 0)),
        pl.BlockSpec((H, C), lambda i: (0, 0)),
        pl.BlockSpec((1, H), lambda i: (0, 0)),
        pl.BlockSpec((1, C), lambda i: (0, 0)),
    ],
    out_specs=pl.BlockSpec((BN, C), lambda i: (i, 0)),
    out_shape=jax.ShapeDtypeStruct((N, C), jnp.float32),
)


@jax.jit
def kernel(x, edge_index, W1, b1, W2, b2, Wfc, bfc):
    row = edge_index[0].astype(jnp.int32).reshape(NW, NCH, CHUNK)
    col = edge_index[1].astype(jnp.int32).reshape(NW, NCH, CHUNK)
    dacc = _deg_kernel(col).reshape(NC, N, H)
    g1 = _tc1(x, W1, dacc)
    acc1 = _prop_kernel(g1, row, col).reshape(NC, N, H)
    g2 = _tc2(acc1, g1, dacc, W2, b1.reshape(1, H))
    acc2 = _prop_kernel(g2, row, col).reshape(NC, N, H)
    return _tc3(acc2, g2, dacc, Wfc, b2.reshape(1, H), bfc.reshape(1, C))
